# baseline (speedup 1.0000x reference)
import functools

import jax
import jax.numpy as jnp
from jax import lax
from jax.experimental import pallas as pl
from jax.experimental.pallas import tpu as pltpu

N_NEIGHBORS = 20
THRESHOLD = -0.5
BIG = 3.0e38


def _enc_body(x_ref, wenc_ref, wcls_ref, emb_ref, preds_ref):
    x = x_ref[...]
    emb = jnp.dot(x, wenc_ref[...], preferred_element_type=jnp.float32)
    emb_ref[...] = emb
    logits = jnp.dot(emb, wcls_ref[...], preferred_element_type=jnp.float32)
    am = jnp.argmax(logits, axis=1).astype(jnp.int32)
    preds_ref[...] = am[:, None]


def _knn_body(emb_ref, keys_ref, ld_ref, idx_ref, topv_ref, topi_ref,
              *, n_chunks, chunk, k_valid):
    i = pl.program_id(0)
    q = emb_ref.shape[0]

    @pl.when(i == 0)
    def _init():
        lane = lax.broadcasted_iota(jnp.int32, (q, 128), 1)
        topv_ref[...] = jnp.where(lane < N_NEIGHBORS,
                                  jnp.full((q, 128), BIG, jnp.float32),
                                  jnp.full((q, 128), -1.0, jnp.float32))
        topi_ref[...] = jnp.zeros((q, 128), jnp.int32)

    emb = emb_ref[...]
    keys = keys_ref[...]
    mm = lax.dot_general(emb, keys, (((1,), (1,)), ((), ())),
                         preferred_element_type=jnp.float32)
    q2 = jnp.sum(emb * emb, axis=1, keepdims=True)
    k2 = jnp.sum(keys * keys, axis=1)
    d2 = q2 - 2.0 * mm + k2[None, :]
    dist = jnp.sqrt(jnp.maximum(d2, 0.0) + 1e-12)
    col = lax.broadcasted_iota(jnp.int32, (q, chunk), 1)
    gcol = i * chunk + col
    dist = jnp.where(gcol < k_valid, dist, BIG)

    lane = lax.broadcasted_iota(jnp.int32, (q, 128), 1)

    topv0 = topv_ref[...]
    topi0 = topi_ref[...]
    worst0 = jnp.max(topv0, axis=1, keepdims=True)
    distc = jnp.where(dist < worst0, dist, BIG)
    cnt = jnp.sum((distc < BIG).astype(jnp.int32), axis=1, keepdims=True)

    def cond(carry):
        c = carry[0]
        return jnp.max(c) > 0

    def body(carry):
        c, dist, topv, topi = carry
        m = jnp.min(dist, axis=1, keepdims=True)
        am = jnp.argmin(dist, axis=1).astype(jnp.int32)
        gi = i * chunk + am
        worst = jnp.max(topv, axis=1, keepdims=True)
        aw = jnp.argmax(topv, axis=1).astype(jnp.int32)
        upd = (lane == aw[:, None]) & (m < worst)
        topv = jnp.where(upd, m, topv)
        topi = jnp.where(upd, gi[:, None], topi)
        dist = jnp.where(col == am[:, None], BIG, dist)
        c = c - (m < BIG).astype(jnp.int32)
        return c, dist, topv, topi

    _, _, topv, topi = lax.while_loop(cond, body,
                                      (cnt, distc, topv0, topi0))
    topv_ref[...] = topv
    topi_ref[...] = topi

    @pl.when(i == n_chunks - 1)
    def _fin():
        tv = jnp.where(lane < N_NEIGHBORS, topv_ref[...], 0.0)
        meanknn = jnp.sum(tv, axis=1, keepdims=True) / N_NEIGHBORS
        ld_ref[...] = 1.0 / (meanknn + 1e-10)
        idx_ref[...] = topi_ref[...]


def kernel(x, keys, key_mean_knn_dist, W_enc, W_cls):
    qn, _ = x.shape
    kn, d = keys.shape
    chunk = 2048
    n_chunks = (kn + chunk - 1) // chunk
    kpad = n_chunks * chunk
    keys_p = jnp.pad(keys, ((0, kpad - kn), (0, 0)))

    emb, preds = pl.pallas_call(
        _enc_body,
        out_shape=(
            jax.ShapeDtypeStruct((qn, d), jnp.float32),
            jax.ShapeDtypeStruct((qn, 1), jnp.int32),
        ),
    )(x, W_enc, W_cls)

    ld, idx = pl.pallas_call(
        functools.partial(_knn_body, n_chunks=n_chunks, chunk=chunk,
                          k_valid=kn),
        grid=(n_chunks,),
        in_specs=[
            pl.BlockSpec((qn, d), lambda i: (0, 0)),
            pl.BlockSpec((chunk, d), lambda i: (i, 0)),
        ],
        out_specs=(
            pl.BlockSpec((qn, 1), lambda i: (0, 0)),
            pl.BlockSpec((qn, 128), lambda i: (0, 0)),
        ),
        out_shape=(
            jax.ShapeDtypeStruct((qn, 1), jnp.float32),
            jax.ShapeDtypeStruct((qn, 128), jnp.int32),
        ),
        scratch_shapes=[
            pltpu.VMEM((qn, 128), jnp.float32),
            pltpu.VMEM((qn, 128), jnp.int32),
        ],
        compiler_params=pltpu.CompilerParams(
            dimension_semantics=("arbitrary",)),
    )(emb, keys_p)

    idx20 = idx[:, :N_NEIGHBORS]
    local_density = ld[:, 0]
    nbr_mean_dist = jnp.take(key_mean_knn_dist, idx20, axis=0)
    influence_density = 1.0 / (nbr_mean_dist + 1e-10)
    avg_influence_density = jnp.mean(influence_density, axis=1)
    inflo_scores = -(local_density / (avg_influence_density + 1e-10))
    ood_flags = inflo_scores < THRESHOLD
    cls_preds = jnp.where(ood_flags, -1, preds[:, 0])
    return ood_flags, cls_preds, inflo_scores


# speedup vs baseline: 5.9709x; 5.9709x over previous
import functools

import jax
import jax.numpy as jnp
from jax import lax
from jax.experimental import pallas as pl
from jax.experimental.pallas import tpu as pltpu

N_NEIGHBORS = 20
THRESHOLD = -0.5
BIG = 3.0e38


def _enc_body(x_ref, wenc_ref, wcls_ref, emb_ref, preds_ref):
    x = x_ref[...]
    emb = jnp.dot(x, wenc_ref[...], preferred_element_type=jnp.float32)
    emb_ref[...] = emb
    logits = jnp.dot(emb, wcls_ref[...], preferred_element_type=jnp.float32)
    am = jnp.argmax(logits, axis=1).astype(jnp.int32)
    preds_ref[...] = am[:, None]


def _knn_body(emb_ref, keys_ref, ld_ref, idx_ref, topv_ref, topi_ref,
              *, n_chunks, chunk, k_valid):
    i = pl.program_id(0)
    q = emb_ref.shape[0]

    @pl.when(i == 0)
    def _init():
        lane = lax.broadcasted_iota(jnp.int32, (q, 128), 1)
        topv_ref[...] = jnp.where(lane < N_NEIGHBORS,
                                  jnp.full((q, 128), BIG, jnp.float32),
                                  jnp.full((q, 128), -1.0, jnp.float32))
        topi_ref[...] = jnp.zeros((q, 128), jnp.int32)

    emb = emb_ref[...]
    keys = keys_ref[...]
    mm = lax.dot_general(emb, keys, (((1,), (1,)), ((), ())),
                         preferred_element_type=jnp.float32)
    q2 = jnp.sum(emb * emb, axis=1, keepdims=True)
    k2 = jnp.sum(keys * keys, axis=1)
    d2 = q2 - 2.0 * mm + k2[None, :]
    dist = jnp.sqrt(jnp.maximum(d2, 0.0) + 1e-12)
    col = lax.broadcasted_iota(jnp.int32, (q, chunk), 1)
    gcol = i * chunk + col
    dist = jnp.where(gcol < k_valid, dist, BIG)

    lane = lax.broadcasted_iota(jnp.int32, (q, 128), 1)

    topv0 = topv_ref[...]
    topi0 = topi_ref[...]
    worst0 = jnp.max(topv0, axis=1, keepdims=True)
    distc = jnp.where(dist < worst0, dist, BIG)
    cnt = jnp.sum((distc < BIG).astype(jnp.int32), axis=1, keepdims=True)
    nmax = jnp.minimum(jnp.max(cnt), N_NEIGHBORS)

    def body(_, carry):
        dist, topv, topi = carry
        m = jnp.min(dist, axis=1, keepdims=True)
        am = jnp.argmin(dist, axis=1).astype(jnp.int32)
        gi = i * chunk + am
        worst = jnp.max(topv, axis=1, keepdims=True)
        aw = jnp.argmax(topv, axis=1).astype(jnp.int32)
        upd = (lane == aw[:, None]) & (m < worst)
        topv = jnp.where(upd, m, topv)
        topi = jnp.where(upd, gi[:, None], topi)
        dist = jnp.where(col == am[:, None], BIG, dist)
        return dist, topv, topi

    _, topv, topi = lax.fori_loop(0, nmax, body, (distc, topv0, topi0))
    topv_ref[...] = topv
    topi_ref[...] = topi

    @pl.when(i == n_chunks - 1)
    def _fin():
        tv = jnp.where(lane < N_NEIGHBORS, topv_ref[...], 0.0)
        meanknn = jnp.sum(tv, axis=1, keepdims=True) / N_NEIGHBORS
        ld_ref[...] = 1.0 / (meanknn + 1e-10)
        idx_ref[...] = topi_ref[...]


def kernel(x, keys, key_mean_knn_dist, W_enc, W_cls):
    qn, _ = x.shape
    kn, d = keys.shape
    chunk = 2048
    n_chunks = (kn + chunk - 1) // chunk
    kpad = n_chunks * chunk
    keys_p = jnp.pad(keys, ((0, kpad - kn), (0, 0)))

    emb, preds = pl.pallas_call(
        _enc_body,
        out_shape=(
            jax.ShapeDtypeStruct((qn, d), jnp.float32),
            jax.ShapeDtypeStruct((qn, 1), jnp.int32),
        ),
    )(x, W_enc, W_cls)

    ld, idx = pl.pallas_call(
        functools.partial(_knn_body, n_chunks=n_chunks, chunk=chunk,
                          k_valid=kn),
        grid=(n_chunks,),
        in_specs=[
            pl.BlockSpec((qn, d), lambda i: (0, 0)),
            pl.BlockSpec((chunk, d), lambda i: (i, 0)),
        ],
        out_specs=(
            pl.BlockSpec((qn, 1), lambda i: (0, 0)),
            pl.BlockSpec((qn, 128), lambda i: (0, 0)),
        ),
        out_shape=(
            jax.ShapeDtypeStruct((qn, 1), jnp.float32),
            jax.ShapeDtypeStruct((qn, 128), jnp.int32),
        ),
        scratch_shapes=[
            pltpu.VMEM((qn, 128), jnp.float32),
            pltpu.VMEM((qn, 128), jnp.int32),
        ],
        compiler_params=pltpu.CompilerParams(
            dimension_semantics=("arbitrary",)),
    )(emb, keys_p)

    idx20 = idx[:, :N_NEIGHBORS]
    local_density = ld[:, 0]
    nbr_mean_dist = jnp.take(key_mean_knn_dist, idx20, axis=0)
    influence_density = 1.0 / (nbr_mean_dist + 1e-10)
    avg_influence_density = jnp.mean(influence_density, axis=1)
    inflo_scores = -(local_density / (avg_influence_density + 1e-10))
    ood_flags = inflo_scores < THRESHOLD
    cls_preds = jnp.where(ood_flags, -1, preds[:, 0])
    return ood_flags, cls_preds, inflo_scores
